# Initial kernel scaffold; baseline (speedup 1.0000x reference)
import jax
import jax.numpy as jnp
from jax.experimental import pallas as pl


def kernel(indices, weight):
    raise NotImplementedError("write your pallas kernel here")


# speedup vs baseline: 1.0419x; 1.0419x over previous
import functools

import jax
import jax.numpy as jnp
from jax.experimental import pallas as pl
from jax.experimental.pallas import tpu as pltpu
from jax.experimental.pallas import tpu_sc as plsc

_B0, _B1 = 16384, 50
_D = 32
_N = _B0 * _B1
_W = 128


def _sc_gather(weight, indices_flat):
    mesh = plsc.VectorSubcoreMesh(
        core_axis_name="core", subcore_axis_name="subcore"
    )

    @functools.partial(
        pl.kernel,
        out_type=jax.ShapeDtypeStruct((_N, _D), weight.dtype),
        mesh=mesh,
        compiler_params=pltpu.CompilerParams(use_tc_tiling_on_sc=False),
    )
    def k(w_hbm, i_hbm, o_hbm):
        def body(i_vmem, o_vmem):
            pltpu.sync_copy(w_hbm.at[i_vmem.at[0]], o_vmem)

        pltpu.emit_pipeline(
            body,
            grid=(_N // _W,),
            in_specs=[pl.BlockSpec((1, _W), index_map=lambda i: (0, i))],
            out_specs=[pl.BlockSpec((_W, _D), index_map=lambda i: (i, 0))],
            core_axis_name=("core", "subcore"),
            dimension_semantics=(pltpu.PARALLEL,),
        )(i_hbm, o_hbm)

    return k(weight, indices_flat)


def kernel(indices, weight):
    out = _sc_gather(weight, indices.reshape(1, _N))
    return out.reshape(_B0, _B1, _D)


# speedup vs baseline: 1.0980x; 1.0538x over previous
import functools

import jax
import jax.numpy as jnp
from jax.experimental import pallas as pl
from jax.experimental.pallas import tpu as pltpu
from jax.experimental.pallas import tpu_sc as plsc

_B0, _B1 = 16384, 50
_D = 32
_N = _B0 * _B1
_W = 512


def _sc_gather(weight, indices_flat):
    mesh = plsc.VectorSubcoreMesh(
        core_axis_name="core", subcore_axis_name="subcore"
    )

    @functools.partial(
        pl.kernel,
        out_type=jax.ShapeDtypeStruct((_N, _D), weight.dtype),
        mesh=mesh,
        compiler_params=pltpu.CompilerParams(use_tc_tiling_on_sc=False),
    )
    def k(w_hbm, i_hbm, o_hbm):
        def body(i_vmem, o_vmem):
            pltpu.sync_copy(w_hbm.at[i_vmem.at[0]], o_vmem)

        pltpu.emit_pipeline(
            body,
            grid=(_N // _W,),
            in_specs=[pl.BlockSpec((1, _W), index_map=lambda i: (0, i))],
            out_specs=[pl.BlockSpec((_W, _D), index_map=lambda i: (i, 0))],
            core_axis_name=("core", "subcore"),
            dimension_semantics=(pltpu.PARALLEL,),
        )(i_hbm, o_hbm)

    return k(weight, indices_flat)


def kernel(indices, weight):
    out = _sc_gather(weight, indices.reshape(1, _N))
    return out.reshape(_B0, _B1, _D)


# speedup vs baseline: 1.4271x; 1.2997x over previous
import functools

import jax
import jax.numpy as jnp
from jax.experimental import pallas as pl
from jax.experimental.pallas import tpu as pltpu
from jax.experimental.pallas import tpu_sc as plsc

_B0, _B1 = 16384, 50
_D = 32
_RB = 16


def kernel(indices, weight):
    mesh = plsc.VectorSubcoreMesh(
        core_axis_name="core", subcore_axis_name="subcore"
    )

    @functools.partial(
        pl.kernel,
        out_type=jax.ShapeDtypeStruct((_B0, _B1, _D), weight.dtype),
        mesh=mesh,
        compiler_params=pltpu.CompilerParams(use_tc_tiling_on_sc=False),
    )
    def k(w_hbm, i_hbm, o_hbm):
        def body(i_vmem, o_vmem):
            for j in range(_RB):
                pltpu.sync_copy(w_hbm.at[i_vmem.at[j]], o_vmem.at[j])

        pltpu.emit_pipeline(
            body,
            grid=(_B0 // _RB,),
            in_specs=[pl.BlockSpec((_RB, _B1), index_map=lambda i: (i, 0))],
            out_specs=[
                pl.BlockSpec((_RB, _B1, _D), index_map=lambda i: (i, 0, 0))
            ],
            core_axis_name=("core", "subcore"),
            dimension_semantics=(pltpu.PARALLEL,),
        )(i_hbm, o_hbm)

    return k(weight, indices)


# speedup vs baseline: 1.7874x; 1.2525x over previous
import functools

import jax
import jax.numpy as jnp
from jax.experimental import pallas as pl
from jax.experimental.pallas import tpu as pltpu
from jax.experimental.pallas import tpu_sc as plsc

_B0, _B1 = 16384, 50
_D = 32
_RB = 16


def kernel(indices, weight):
    mesh = plsc.VectorSubcoreMesh(
        core_axis_name="core", subcore_axis_name="subcore"
    )

    @functools.partial(
        pl.kernel,
        out_type=jax.ShapeDtypeStruct((_B0, _B1, _D), weight.dtype),
        mesh=mesh,
        scratch_types=[pltpu.SemaphoreType.DMA],
        compiler_params=pltpu.CompilerParams(use_tc_tiling_on_sc=False),
    )
    def k(w_hbm, i_hbm, o_hbm, sem):
        def body(i_vmem, o_vmem):
            descs = [
                pltpu.async_copy(w_hbm.at[i_vmem.at[j]], o_vmem.at[j], sem)
                for j in range(_RB)
            ]
            for d in descs:
                d.wait()

        pltpu.emit_pipeline(
            body,
            grid=(_B0 // _RB,),
            in_specs=[pl.BlockSpec((_RB, _B1), index_map=lambda i: (i, 0))],
            out_specs=[
                pl.BlockSpec((_RB, _B1, _D), index_map=lambda i: (i, 0, 0))
            ],
            core_axis_name=("core", "subcore"),
            dimension_semantics=(pltpu.PARALLEL,),
        )(i_hbm, o_hbm)

    return k(weight, indices)
